# baseline (speedup 1.0000x reference)
import functools

import jax
import jax.numpy as jnp
from jax import lax
from jax.experimental import pallas as pl
from jax.experimental.pallas import tpu as pltpu
from jax.experimental.pallas import tpu_sc as plsc

N = 10000
E = 320000
IN = 128
HID = 128
OUT = 64

NC = 2
NS = 16
NW = NC * NS
CH = 128
NPAD = 10240
ROWS_PER_TILE = NPAD // NS
NCHUNK = 80
EPAD = NW * NCHUNK * CH
DW = 128

_sc_mesh = plsc.VectorSubcoreMesh(
    core_axis_name="c", subcore_axis_name="s", num_cores=NC, num_subcores=NS
)


@functools.partial(
    pl.kernel,
    out_type=jax.ShapeDtypeStruct((NC, NPAD, DW), jnp.float32),
    mesh=_sc_mesh,
    scratch_types=[
        pltpu.VMEM((NCHUNK + 1, 2, CH), jnp.int32),
        pltpu.VMEM((CH, DW), jnp.float32),
        pltpu.VMEM_SHARED((NPAD, DW), jnp.float32),
    ],
)
def _deg_kernel(edges, ones_hbm, zeros_hbm, out, ei, ones_v, acc):
    cid = lax.axis_index("c")
    sid = lax.axis_index("s")
    wid = cid * NS + sid
    pltpu.sync_copy(zeros_hbm, acc.at[pl.ds(sid * ROWS_PER_TILE, ROWS_PER_TILE)])
    pltpu.sync_copy(ones_hbm, ones_v)
    pltpu.sync_copy(edges.at[wid], ei)
    plsc.subcore_barrier()

    def body(k, carry):
        pltpu.sync_copy(ones_v, acc.at[ei.at[k, 1]], add=True)
        return carry

    lax.fori_loop(0, NCHUNK, body, 0)
    plsc.subcore_barrier()
    sl = pl.ds(sid * ROWS_PER_TILE, ROWS_PER_TILE)
    pltpu.sync_copy(acc.at[sl], out.at[cid, sl])


@functools.partial(
    pl.kernel,
    out_type=jax.ShapeDtypeStruct((NC, NPAD, HID), jnp.float32),
    mesh=_sc_mesh,
    scratch_types=[
        pltpu.VMEM((2, CH), jnp.int32),
        pltpu.VMEM((2, CH), jnp.int32),
        pltpu.VMEM((CH, HID), jnp.float32),
        pltpu.VMEM((CH, HID), jnp.float32),
        pltpu.VMEM_SHARED((NPAD, HID), jnp.float32),
        pltpu.SemaphoreType.DMA,
        pltpu.SemaphoreType.DMA,
        pltpu.SemaphoreType.DMA,
        pltpu.SemaphoreType.DMA,
    ],
)
def _agg_kernel(table, edges, zeros_hbm, out, eia, eib, ra, rb, acc,
                sa, sb, ea, eb):
    cid = lax.axis_index("c")
    sid = lax.axis_index("s")
    wid = cid * NS + sid
    pltpu.sync_copy(zeros_hbm, acc.at[pl.ds(sid * ROWS_PER_TILE, ROWS_PER_TILE)])
    plsc.subcore_barrier()

    pltpu.sync_copy(edges.at[wid, 0], eia)
    pltpu.async_copy(table.at[eia.at[0]], ra, sa)
    pltpu.async_copy(edges.at[wid, 1], eib, eb)

    def body(j, carry):
        k = 2 * j
        pltpu.make_async_copy(edges.at[wid, k + 1], eib, eb).wait()
        cpb = pltpu.async_copy(table.at[eib.at[0]], rb, sb)
        pltpu.make_async_copy(table.at[eia.at[0]], ra, sa).wait()
        pltpu.sync_copy(ra, acc.at[eia.at[1]], add=True)
        pltpu.async_copy(edges.at[wid, k + 2], eia, ea)
        pltpu.make_async_copy(edges.at[wid, k + 2], eia, ea).wait()
        pltpu.async_copy(table.at[eia.at[0]], ra, sa)
        cpb.wait()
        pltpu.sync_copy(rb, acc.at[eib.at[1]], add=True)
        pltpu.async_copy(edges.at[wid, k + 3], eib, eb)
        return carry

    lax.fori_loop(0, NCHUNK // 2, body, 0)
    pltpu.make_async_copy(table.at[eia.at[0]], ra, sa).wait()
    pltpu.make_async_copy(edges.at[wid, NCHUNK + 1], eib, eb).wait()
    plsc.subcore_barrier()
    sl = pl.ds(sid * ROWS_PER_TILE, ROWS_PER_TILE)
    pltpu.sync_copy(acc.at[sl], out.at[cid, sl])


_R = 1024
_GRID = NPAD // _R


def _row_spec(w):
    return pl.BlockSpec((_R, w), lambda i: (i, 0))


def _full_spec(shape):
    nd = len(shape)
    return pl.BlockSpec(shape, lambda i: (0,) * nd)


def _dinv(d0, d1):
    return lax.rsqrt(1.0 + d0[:, :1] + d1[:, :1])


def _tc1_body(x_ref, we_ref, be_ref, w1_ref, ego_ref, g1_ref):
    xb = x_ref[...]
    ego_ref[...] = jnp.maximum(
        jnp.dot(xb, we_ref[...], preferred_element_type=jnp.float32) + be_ref[...],
        0.0,
    )
    g1_ref[...] = jnp.dot(xb, w1_ref[...], preferred_element_type=jnp.float32)


def _tc2_body(d0_ref, d1_ref, g1_ref, g1s_ref):
    g1s_ref[...] = _dinv(d0_ref[...], d1_ref[...]) * g1_ref[...]


def _tc3_body(d0_ref, d1_ref, a0_ref, a1_ref, g1s_ref, b1_ref, w2_ref,
              h1_ref, g2s_ref):
    dinv = _dinv(d0_ref[...], d1_ref[...])
    s1 = dinv * (a0_ref[...] + a1_ref[...] + g1s_ref[...]) + b1_ref[...]
    h1 = jnp.maximum(s1, 0.0)
    h1_ref[...] = h1
    g2s_ref[...] = dinv * jnp.dot(h1, w2_ref[...], preferred_element_type=jnp.float32)


def _tc4_body(d0_ref, d1_ref, a0_ref, a1_ref, g2s_ref, b2_ref,
              ego_ref, h1_ref, wce_ref, wc1_ref, wc2_ref, bc_ref, out_ref):
    dinv = _dinv(d0_ref[...], d1_ref[...])
    s2 = dinv * (a0_ref[...] + a1_ref[...] + g2s_ref[...]) + b2_ref[...]
    h2 = jnp.maximum(s2, 0.0)
    acc = jnp.dot(ego_ref[...], wce_ref[...], preferred_element_type=jnp.float32)
    acc += jnp.dot(h1_ref[...], wc1_ref[...], preferred_element_type=jnp.float32)
    acc += jnp.dot(h2, wc2_ref[...], preferred_element_type=jnp.float32)
    out_ref[...] = acc + bc_ref[...]


def _out2(w):
    return jax.ShapeDtypeStruct((NPAD, w), jnp.float32)


def kernel(x, edge_index, W_ego, b_ego, W1, b1, W2, b2, Wc, bc):
    f32 = jnp.float32
    x_pad = jnp.pad(x, ((0, NPAD - N), (0, 0)))

    pad_e = EPAD - E
    src = jnp.concatenate([edge_index[0], jnp.full((pad_e,), NPAD - 1, jnp.int32)])
    dst = jnp.concatenate([edge_index[1], jnp.full((pad_e,), NPAD - 1, jnp.int32)])
    dummy = jnp.full((NW, 1, CH), NPAD - 1, jnp.int32)
    edges = jnp.stack(
        [
            jnp.concatenate([src.reshape(NW, NCHUNK, CH), dummy], axis=1),
            jnp.concatenate([dst.reshape(NW, NCHUNK, CH), dummy], axis=1),
        ],
        axis=2,
    )

    ones_d = jnp.ones((CH, DW), f32)
    zeros_d = jnp.zeros((ROWS_PER_TILE, DW), f32)

    deg_parts = _deg_kernel(edges, ones_d, zeros_d)
    d0, d1 = deg_parts[0], deg_parts[1]

    ego, g1 = pl.pallas_call(
        _tc1_body,
        grid=(_GRID,),
        in_specs=[
            _row_spec(IN),
            _full_spec((IN, HID)),
            _full_spec((1, HID)),
            _full_spec((IN, HID)),
        ],
        out_specs=[_row_spec(HID), _row_spec(HID)],
        out_shape=[_out2(HID), _out2(HID)],
    )(x_pad, W_ego, b_ego.reshape(1, HID), W1)

    g1s = pl.pallas_call(
        _tc2_body,
        grid=(_GRID,),
        in_specs=[_row_spec(DW), _row_spec(DW), _row_spec(HID)],
        out_specs=_row_spec(HID),
        out_shape=_out2(HID),
    )(d0, d1, g1)

    agg1 = _agg_kernel(g1s, edges, zeros_d)

    h1, g2s = pl.pallas_call(
        _tc3_body,
        grid=(_GRID,),
        in_specs=[
            _row_spec(DW), _row_spec(DW),
            _row_spec(HID), _row_spec(HID), _row_spec(HID),
            _full_spec((1, HID)), _full_spec((HID, HID)),
        ],
        out_specs=[_row_spec(HID), _row_spec(HID)],
        out_shape=[_out2(HID), _out2(HID)],
    )(d0, d1, agg1[0], agg1[1], g1s, b1.reshape(1, HID), W2)

    agg2 = _agg_kernel(g2s, edges, zeros_d)

    out = pl.pallas_call(
        _tc4_body,
        grid=(_GRID,),
        in_specs=[
            _row_spec(DW), _row_spec(DW),
            _row_spec(HID), _row_spec(HID), _row_spec(HID),
            _full_spec((1, HID)),
            _row_spec(HID), _row_spec(HID),
            _full_spec((HID, OUT)), _full_spec((HID, OUT)), _full_spec((HID, OUT)),
            _full_spec((1, OUT)),
        ],
        out_specs=_row_spec(OUT),
        out_shape=_out2(OUT),
    )(
        d0, d1, agg2[0], agg2[1], g2s, b2.reshape(1, HID),
        ego, h1, Wc[:HID], Wc[HID:2 * HID], Wc[2 * HID:], bc.reshape(1, OUT),
    )

    return out[:N]


# speedup vs baseline: 1.5281x; 1.5281x over previous
import functools

import jax
import jax.numpy as jnp
from jax import lax
from jax.experimental import pallas as pl
from jax.experimental.pallas import tpu as pltpu
from jax.experimental.pallas import tpu_sc as plsc

N = 10000
E = 320000
IN = 128
HID = 128
OUT = 64

NC = 2
NS = 16
NW = NC * NS
CH = 64
NPAD = 10240
ROWS_PER_TILE = NPAD // NS
NCHUNK = 158
EPAD = NW * NCHUNK * CH
DW = 128

_sc_mesh = plsc.VectorSubcoreMesh(
    core_axis_name="c", subcore_axis_name="s", num_cores=NC, num_subcores=NS
)


@functools.partial(
    pl.kernel,
    out_type=jax.ShapeDtypeStruct((NC, NPAD, DW), jnp.float32),
    mesh=_sc_mesh,
    scratch_types=[
        pltpu.VMEM((NCHUNK + 1, 2 * CH), jnp.int32),
        pltpu.VMEM((CH, DW), jnp.float32),
        pltpu.VMEM_SHARED((NPAD, DW), jnp.float32),
    ],
)
def _deg_kernel(edges, ones_hbm, zeros_hbm, out, ei, ones_v, acc):
    cid = lax.axis_index("c")
    sid = lax.axis_index("s")
    wid = cid * NS + sid
    pltpu.sync_copy(zeros_hbm, acc.at[pl.ds(sid * ROWS_PER_TILE, ROWS_PER_TILE)])
    pltpu.sync_copy(ones_hbm, ones_v)
    pltpu.sync_copy(edges.at[wid], ei)
    plsc.subcore_barrier()

    def body(k, carry):
        pltpu.sync_copy(ones_v, acc.at[ei.at[k, pl.ds(CH, CH)]], add=True)
        return carry

    lax.fori_loop(0, NCHUNK, body, 0)
    plsc.subcore_barrier()
    sl = pl.ds(sid * ROWS_PER_TILE, ROWS_PER_TILE)
    pltpu.sync_copy(acc.at[sl], out.at[cid, sl])


@functools.partial(
    pl.kernel,
    out_type=jax.ShapeDtypeStruct((NC, NPAD, HID), jnp.float32),
    mesh=_sc_mesh,
    scratch_types=[
        pltpu.VMEM((NCHUNK + 1, 2 * CH), jnp.int32),
        pltpu.VMEM((CH, HID), jnp.float32),
        pltpu.VMEM((CH, HID), jnp.float32),
        pltpu.VMEM_SHARED((NPAD, HID), jnp.float32),
        pltpu.SemaphoreType.DMA,
        pltpu.SemaphoreType.DMA,
    ],
)
def _agg_kernel(table, edges, zeros_hbm, out, ei, ra, rb, acc, sa, sb):
    cid = lax.axis_index("c")
    sid = lax.axis_index("s")
    wid = cid * NS + sid
    pltpu.sync_copy(zeros_hbm, acc.at[pl.ds(sid * ROWS_PER_TILE, ROWS_PER_TILE)])
    pltpu.sync_copy(edges.at[wid], ei)
    plsc.subcore_barrier()

    def _src(k):
        return ei.at[k, pl.ds(0, CH)]

    def _dst(k):
        return ei.at[k, pl.ds(CH, CH)]

    pltpu.async_copy(table.at[_src(0)], ra, sa)

    def body(j, carry):
        k = 2 * j
        cpb = pltpu.async_copy(table.at[_src(k + 1)], rb, sb)
        pltpu.make_async_copy(table.at[_src(k)], ra, sa).wait()
        pltpu.sync_copy(ra, acc.at[_dst(k)], add=True)
        pltpu.async_copy(table.at[_src(k + 2)], ra, sa)
        cpb.wait()
        pltpu.sync_copy(rb, acc.at[_dst(k + 1)], add=True)
        return carry

    lax.fori_loop(0, NCHUNK // 2, body, 0)
    pltpu.make_async_copy(table.at[_src(NCHUNK)], ra, sa).wait()
    plsc.subcore_barrier()
    sl = pl.ds(sid * ROWS_PER_TILE, ROWS_PER_TILE)
    pltpu.sync_copy(acc.at[sl], out.at[cid, sl])


_R = 1024
_GRID = NPAD // _R


def _row_spec(w):
    return pl.BlockSpec((_R, w), lambda i: (i, 0))


def _full_spec(shape):
    nd = len(shape)
    return pl.BlockSpec(shape, lambda i: (0,) * nd)


def _dinv(d0, d1):
    return lax.rsqrt(1.0 + d0[:, :1] + d1[:, :1])


def _tc1_body(x_ref, we_ref, be_ref, w1_ref, ego_ref, g1_ref):
    xb = x_ref[...]
    ego_ref[...] = jnp.maximum(
        jnp.dot(xb, we_ref[...], preferred_element_type=jnp.float32) + be_ref[...],
        0.0,
    )
    g1_ref[...] = jnp.dot(xb, w1_ref[...], preferred_element_type=jnp.float32)


def _tc2_body(d0_ref, d1_ref, g1_ref, g1s_ref):
    g1s_ref[...] = _dinv(d0_ref[...], d1_ref[...]) * g1_ref[...]


def _tc3_body(d0_ref, d1_ref, a0_ref, a1_ref, g1s_ref, b1_ref, w2_ref,
              h1_ref, g2s_ref):
    dinv = _dinv(d0_ref[...], d1_ref[...])
    s1 = dinv * (a0_ref[...] + a1_ref[...] + g1s_ref[...]) + b1_ref[...]
    h1 = jnp.maximum(s1, 0.0)
    h1_ref[...] = h1
    g2s_ref[...] = dinv * jnp.dot(h1, w2_ref[...], preferred_element_type=jnp.float32)


def _tc4_body(d0_ref, d1_ref, a0_ref, a1_ref, g2s_ref, b2_ref,
              ego_ref, h1_ref, wce_ref, wc1_ref, wc2_ref, bc_ref, out_ref):
    dinv = _dinv(d0_ref[...], d1_ref[...])
    s2 = dinv * (a0_ref[...] + a1_ref[...] + g2s_ref[...]) + b2_ref[...]
    h2 = jnp.maximum(s2, 0.0)
    acc = jnp.dot(ego_ref[...], wce_ref[...], preferred_element_type=jnp.float32)
    acc += jnp.dot(h1_ref[...], wc1_ref[...], preferred_element_type=jnp.float32)
    acc += jnp.dot(h2, wc2_ref[...], preferred_element_type=jnp.float32)
    out_ref[...] = acc + bc_ref[...]


def _out2(w):
    return jax.ShapeDtypeStruct((NPAD, w), jnp.float32)


def kernel(x, edge_index, W_ego, b_ego, W1, b1, W2, b2, Wc, bc):
    f32 = jnp.float32
    x_pad = jnp.pad(x, ((0, NPAD - N), (0, 0)))

    pad_e = EPAD - E
    src = jnp.concatenate([edge_index[0], jnp.full((pad_e,), NPAD - 1, jnp.int32)])
    dst = jnp.concatenate([edge_index[1], jnp.full((pad_e,), NPAD - 1, jnp.int32)])
    dummy = jnp.full((NW, 1, 2 * CH), NPAD - 1, jnp.int32)
    packed = jnp.concatenate(
        [src.reshape(NW, NCHUNK, CH), dst.reshape(NW, NCHUNK, CH)], axis=2
    )
    edges = jnp.concatenate([packed, dummy], axis=1)

    ones_d = jnp.ones((CH, DW), f32)
    zeros_d = jnp.zeros((ROWS_PER_TILE, DW), f32)

    deg_parts = _deg_kernel(edges, ones_d, zeros_d)
    d0, d1 = deg_parts[0], deg_parts[1]

    ego, g1 = pl.pallas_call(
        _tc1_body,
        grid=(_GRID,),
        in_specs=[
            _row_spec(IN),
            _full_spec((IN, HID)),
            _full_spec((1, HID)),
            _full_spec((IN, HID)),
        ],
        out_specs=[_row_spec(HID), _row_spec(HID)],
        out_shape=[_out2(HID), _out2(HID)],
    )(x_pad, W_ego, b_ego.reshape(1, HID), W1)

    g1s = pl.pallas_call(
        _tc2_body,
        grid=(_GRID,),
        in_specs=[_row_spec(DW), _row_spec(DW), _row_spec(HID)],
        out_specs=_row_spec(HID),
        out_shape=_out2(HID),
    )(d0, d1, g1)

    agg1 = _agg_kernel(g1s, edges, zeros_d)

    h1, g2s = pl.pallas_call(
        _tc3_body,
        grid=(_GRID,),
        in_specs=[
            _row_spec(DW), _row_spec(DW),
            _row_spec(HID), _row_spec(HID), _row_spec(HID),
            _full_spec((1, HID)), _full_spec((HID, HID)),
        ],
        out_specs=[_row_spec(HID), _row_spec(HID)],
        out_shape=[_out2(HID), _out2(HID)],
    )(d0, d1, agg1[0], agg1[1], g1s, b1.reshape(1, HID), W2)

    agg2 = _agg_kernel(g2s, edges, zeros_d)

    out = pl.pallas_call(
        _tc4_body,
        grid=(_GRID,),
        in_specs=[
            _row_spec(DW), _row_spec(DW),
            _row_spec(HID), _row_spec(HID), _row_spec(HID),
            _full_spec((1, HID)),
            _row_spec(HID), _row_spec(HID),
            _full_spec((HID, OUT)), _full_spec((HID, OUT)), _full_spec((HID, OUT)),
            _full_spec((1, OUT)),
        ],
        out_specs=_row_spec(OUT),
        out_shape=_out2(OUT),
    )(
        d0, d1, agg2[0], agg2[1], g2s, b2.reshape(1, HID),
        ego, h1, Wc[:HID], Wc[HID:2 * HID], Wc[2 * HID:], bc.reshape(1, OUT),
    )

    return out[:N]
